# baseline (speedup 1.0000x reference)
import jax
import jax.numpy as jnp
from jax import lax
from jax.experimental import pallas as pl
from jax.experimental.pallas import tpu as pltpu
from jax.experimental.pallas import tpu_sc as plsc

ZCH = 1_000_000
PADDED = 1 << 20
NUM_IDS = 819_200
NC, NS, L = 2, 16, 16
NW = NC * NS
PER_TILE = NUM_IDS // NW
CHUNK = 1280
NCHUNK = PER_TILE // CHUNK
VPC = CHUNK // L
NSPLIT = PADDED // 16
NB = 16384
EPT = NB // NS
MROW = (ZCH - 1) // 16
I32MAX = 2**31 - 1


def _i32(x):
    return jnp.int32(x)


def _body(values_hbm, fused_hbm, splitter_hbm, out_hbm,
          splitter_v, off_v, offpart_v, q_v, lo_v, hi_v, gbuf_v,
          off_shared, sem1):
    wid = lax.axis_index("c") * NS + lax.axis_index("s")
    sid = lax.axis_index("s")
    pltpu.sync_copy(splitter_hbm, splitter_v)
    iota = lax.iota(jnp.int32, L)

    @plsc.parallel_loop(0, EPT // L, unroll=4)
    def _edges(j):
        ji = j.astype(jnp.int32) * _i32(L) + iota
        e = (sid * _i32(EPT) + ji) << _i32(17)
        s = jnp.zeros((L,), jnp.int32)
        for k in range(15, -1, -1):
            cur = 1 << k
            t = plsc.load_gather(splitter_v, [s + _i32(cur - 1)])
            s = s + jnp.where(t < e, _i32(cur), _i32(0))
        plsc.store_scatter(offpart_v, [ji], s)

    pltpu.sync_copy(offpart_v, off_shared.at[pl.ds(sid * _i32(EPT), EPT)])
    plsc.subcore_barrier()
    pltpu.sync_copy(off_shared, off_v.at[pl.ds(0, NB)])
    plsc.store_scatter(off_v, [_i32(NB) + iota],
                       jnp.full((L,), NSPLIT - 1, jnp.int32),
                       mask=iota < _i32(2))

    @pl.loop(0, NCHUNK)
    def _chunk(c):
        base = wid * _i32(PER_TILE) + c * _i32(CHUNK)
        pltpu.sync_copy(values_hbm.at[pl.ds(base, CHUNK)], q_v)

        def _sweep0(i, wmax):
            qi = i.astype(jnp.int32) * _i32(L) + iota
            v = plsc.load_gather(q_v, [qi])
            b = v >> _i32(17)
            lo = plsc.load_gather(off_v, [b])
            hi = plsc.load_gather(off_v, [b + _i32(1)])
            plsc.store_scatter(lo_v, [qi], lo)
            plsc.store_scatter(hi_v, [qi], hi)
            return jnp.maximum(wmax, hi - lo)

        wmax = plsc.parallel_loop(
            0, VPC, carry=jnp.zeros((L,), jnp.int32))(_sweep0)
        w0 = jnp.max(wmax)

        def _lvl_body(w):
            @plsc.parallel_loop(0, VPC, unroll=4)
            def _lvl(i):
                qi = i.astype(jnp.int32) * _i32(L) + iota
                v = plsc.load_gather(q_v, [qi])
                lo = plsc.load_gather(lo_v, [qi])
                hi = plsc.load_gather(hi_v, [qi])
                act = hi > lo
                mid = (lo + hi) >> _i32(1)
                t = plsc.load_gather(splitter_v, [mid])
                lt = t < v
                plsc.store_scatter(
                    lo_v, [qi], jnp.where(act & lt, mid + _i32(1), lo))
                plsc.store_scatter(
                    hi_v, [qi], jnp.where(act & (~lt), mid, hi))
            return w >> _i32(1)

        lax.while_loop(lambda w: w > _i32(0), _lvl_body, w0)

        pltpu.async_copy(fused_hbm.at[lo_v], gbuf_v, sem1).wait()

        @plsc.parallel_loop(0, VPC, unroll=4)
        def _sweep2(i):
            qi = i.astype(jnp.int32) * _i32(L) + iota
            v = plsc.load_gather(q_v, [qi])
            o = jnp.zeros((L,), jnp.int32)
            for k in range(3, -1, -1):
                cur = 1 << k
                t = plsc.load_gather(gbuf_v, [qi, o + _i32(cur - 1)])
                o = o + jnp.where(t < v, _i32(cur), _i32(0))
            r = plsc.load_gather(gbuf_v, [qi, o])
            m = plsc.load_gather(gbuf_v, [qi, o + _i32(16)])
            plsc.store_scatter(hi_v, [qi], jnp.where(r == v, m, _i32(ZCH - 1)))

        pltpu.sync_copy(hi_v, out_hbm.at[pl.ds(base, CHUNK)])


def _run(v32, fused2d, splitter):
    with jax.enable_x64(False):
        mesh = plsc.VectorSubcoreMesh(core_axis_name="c", subcore_axis_name="s")
        return pl.kernel(
            _body,
            out_type=jax.ShapeDtypeStruct((NUM_IDS,), jnp.int32),
            mesh=mesh,
            scratch_types=[
                pltpu.VMEM((NSPLIT,), jnp.int32),
                pltpu.VMEM((NB + 2,), jnp.int32),
                pltpu.VMEM((EPT,), jnp.int32),
                pltpu.VMEM((CHUNK,), jnp.int32),
                pltpu.VMEM((CHUNK,), jnp.int32),
                pltpu.VMEM((CHUNK,), jnp.int32),
                pltpu.VMEM((CHUNK, 32), jnp.int32),
                pltpu.VMEM_SHARED((NB,), jnp.int32),
                pltpu.SemaphoreType.DMA,
            ],
            compiler_params=pltpu.CompilerParams(
                needs_layout_passes=False, use_tc_tiling_on_sc=False),
        )(v32, fused2d, splitter)


def kernel(values, mch_sorted_raw_ids, mch_remapped_ids_mapping):
    v32 = values.astype(jnp.int32)
    t32 = mch_sorted_raw_ids.astype(jnp.int32)
    m32 = mch_remapped_ids_mapping.astype(jnp.int32)
    pad = jnp.full((PADDED - ZCH,), I32MAX, jnp.int32)
    table2d = jnp.concatenate([t32, pad]).reshape(PADDED // 16, 16)
    splitter = table2d[:, 15]
    mapping2d = jnp.concatenate(
        [m32, jnp.zeros((PADDED - ZCH,), jnp.int32)]).reshape(PADDED // 16, 16)
    fused2d = jnp.concatenate([table2d, mapping2d], axis=1)
    out32 = _run(v32, fused2d, splitter)
    return out32.astype(mch_remapped_ids_mapping.dtype)


# speedup vs baseline: 1.4354x; 1.4354x over previous
import jax
import jax.numpy as jnp
from jax import lax
from jax.experimental import pallas as pl
from jax.experimental.pallas import tpu as pltpu
from jax.experimental.pallas import tpu_sc as plsc

ZCH = 1_000_000
PADDED = 1 << 20
NUM_IDS = 819_200
NC, NS, L = 2, 16, 16
NW = NC * NS
PER_TILE = NUM_IDS // NW
CHUNK = 1024
NCHUNK = PER_TILE // CHUNK
VPC = CHUNK // L
NSPLIT = PADDED // 16
NB = 16384
EPT = NB // NS
MROW = (ZCH - 1) // 16
I32MAX = 2**31 - 1


def _i32(x):
    return jnp.int32(x)


def _body(values_hbm, table2d_hbm, splitter_hbm, mapping2d_hbm, out_hbm,
          splitter_v, off_v, offpart_v, q0_v, q1_v, lo0_v, lo1_v,
          row0_v, row1_v, hi_v, gbuf_v, mbuf_v, off_shared, sem1, sem2):
    wid = lax.axis_index("c") * NS + lax.axis_index("s")
    sid = lax.axis_index("s")
    pltpu.sync_copy(splitter_hbm, splitter_v)
    iota = lax.iota(jnp.int32, L)

    @plsc.parallel_loop(0, EPT // L, unroll=4)
    def _edges(j):
        ji = j.astype(jnp.int32) * _i32(L) + iota
        e = (sid * _i32(EPT) + ji) << _i32(17)
        s = jnp.zeros((L,), jnp.int32)
        for k in range(15, -1, -1):
            cur = 1 << k
            t = plsc.load_gather(splitter_v, [s + _i32(cur - 1)])
            s = s + jnp.where(t < e, _i32(cur), _i32(0))
        plsc.store_scatter(offpart_v, [ji], s)

    pltpu.sync_copy(offpart_v, off_shared.at[pl.ds(sid * _i32(EPT), EPT)])
    plsc.subcore_barrier()
    pltpu.sync_copy(off_shared, off_v.at[pl.ds(0, NB)])
    plsc.store_scatter(off_v, [_i32(NB) + iota],
                       jnp.full((L,), NSPLIT - 1, jnp.int32),
                       mask=iota < _i32(2))

    def _base(c):
        return wid * _i32(PER_TILE) + c * _i32(CHUNK)

    def _stage_a(c, q_b, lo_b, row_b):
        pltpu.sync_copy(values_hbm.at[pl.ds(_base(c), CHUNK)], q_b)

        def _sweep0(i, wmax):
            qi = i.astype(jnp.int32) * _i32(L) + iota
            v = plsc.load_gather(q_b, [qi])
            b = v >> _i32(17)
            lo = plsc.load_gather(off_v, [b])
            hi = plsc.load_gather(off_v, [b + _i32(1)])
            plsc.store_scatter(lo_b, [qi], lo)
            plsc.store_scatter(hi_v, [qi], hi)
            return jnp.maximum(wmax, hi - lo)

        wmax = plsc.parallel_loop(
            0, VPC, carry=jnp.zeros((L,), jnp.int32))(_sweep0)
        w0 = jnp.max(wmax)

        def _lvl_body(w):
            @plsc.parallel_loop(0, VPC, unroll=4)
            def _lvl(i):
                qi = i.astype(jnp.int32) * _i32(L) + iota
                v = plsc.load_gather(q_b, [qi])
                lo = plsc.load_gather(lo_b, [qi])
                hi = plsc.load_gather(hi_v, [qi])
                act = hi > lo
                mid = (lo + hi) >> _i32(1)
                t = plsc.load_gather(splitter_v, [mid])
                lt = t < v
                plsc.store_scatter(
                    lo_b, [qi], jnp.where(act & lt, mid + _i32(1), lo))
                plsc.store_scatter(
                    hi_v, [qi], jnp.where(act & (~lt), mid, hi))
            return w >> _i32(1)

        lax.while_loop(lambda w: w > _i32(0), _lvl_body, w0)

        @plsc.parallel_loop(0, VPC, unroll=4)
        def _rows(i):
            qi = i.astype(jnp.int32) * _i32(L) + iota
            s = plsc.load_gather(lo_b, [qi])
            plsc.store_scatter(row_b, [qi], jnp.minimum(s, _i32(MROW)))

    def _dma_start(lo_b, row_b):
        pltpu.async_copy(table2d_hbm.at[lo_b], gbuf_v, sem1)
        pltpu.async_copy(mapping2d_hbm.at[row_b], mbuf_v, sem2)

    def _finish(c, q_b, lo_b, row_b):
        pltpu.make_async_copy(table2d_hbm.at[lo_b], gbuf_v, sem1).wait()
        pltpu.make_async_copy(mapping2d_hbm.at[row_b], mbuf_v, sem2).wait()

        @plsc.parallel_loop(0, VPC, unroll=4)
        def _sweep2(i):
            qi = i.astype(jnp.int32) * _i32(L) + iota
            v = plsc.load_gather(q_b, [qi])
            s = plsc.load_gather(lo_b, [qi])
            o = jnp.zeros((L,), jnp.int32)
            for k in range(3, -1, -1):
                cur = 1 << k
                t = plsc.load_gather(gbuf_v, [qi, o + _i32(cur - 1)])
                o = o + jnp.where(t < v, _i32(cur), _i32(0))
            r = plsc.load_gather(gbuf_v, [qi, o])
            searched = jnp.minimum(s * _i32(16) + o, _i32(ZCH - 1))
            m = plsc.load_gather(mbuf_v, [qi, searched & _i32(15)])
            plsc.store_scatter(hi_v, [qi], jnp.where(r == v, m, _i32(ZCH - 1)))

        pltpu.sync_copy(hi_v, out_hbm.at[pl.ds(_base(c), CHUNK)])

    _stage_a(_i32(0), q0_v, lo0_v, row0_v)
    _dma_start(lo0_v, row0_v)

    @pl.loop(0, (NCHUNK - 1) // 2)
    def _pair(d):
        di = d.astype(jnp.int32)
        c1 = di * _i32(2) + _i32(1)
        c2 = c1 + _i32(1)
        _stage_a(c1, q1_v, lo1_v, row1_v)
        _finish(di * _i32(2), q0_v, lo0_v, row0_v)
        _dma_start(lo1_v, row1_v)
        _stage_a(c2, q0_v, lo0_v, row0_v)
        _finish(c1, q1_v, lo1_v, row1_v)
        _dma_start(lo0_v, row0_v)

    _finish(_i32(NCHUNK - 1), q0_v, lo0_v, row0_v)


def _run(v32, table2d, splitter, mapping2d):
    with jax.enable_x64(False):
        mesh = plsc.VectorSubcoreMesh(core_axis_name="c", subcore_axis_name="s")
        return pl.kernel(
            _body,
            out_type=jax.ShapeDtypeStruct((NUM_IDS,), jnp.int32),
            mesh=mesh,
            scratch_types=[
                pltpu.VMEM((NSPLIT,), jnp.int32),
                pltpu.VMEM((NB + 2,), jnp.int32),
                pltpu.VMEM((EPT,), jnp.int32),
                pltpu.VMEM((CHUNK,), jnp.int32),
                pltpu.VMEM((CHUNK,), jnp.int32),
                pltpu.VMEM((CHUNK,), jnp.int32),
                pltpu.VMEM((CHUNK,), jnp.int32),
                pltpu.VMEM((CHUNK,), jnp.int32),
                pltpu.VMEM((CHUNK,), jnp.int32),
                pltpu.VMEM((CHUNK,), jnp.int32),
                pltpu.VMEM((CHUNK, 16), jnp.int32),
                pltpu.VMEM((CHUNK, 16), jnp.int32),
                pltpu.VMEM_SHARED((NB,), jnp.int32),
                pltpu.SemaphoreType.DMA,
                pltpu.SemaphoreType.DMA,
            ],
            compiler_params=pltpu.CompilerParams(
                needs_layout_passes=False, use_tc_tiling_on_sc=False),
        )(v32, table2d, splitter, mapping2d)


def kernel(values, mch_sorted_raw_ids, mch_remapped_ids_mapping):
    v32 = values.astype(jnp.int32)
    t32 = mch_sorted_raw_ids.astype(jnp.int32)
    m32 = mch_remapped_ids_mapping.astype(jnp.int32)
    pad = jnp.full((PADDED - ZCH,), I32MAX, jnp.int32)
    table2d = jnp.concatenate([t32, pad]).reshape(PADDED // 16, 16)
    splitter = table2d[:, 15]
    mapping2d = m32.reshape(ZCH // 16, 16)
    out32 = _run(v32, table2d, splitter, mapping2d)
    return out32.astype(mch_remapped_ids_mapping.dtype)
